# baseline (speedup 1.0000x reference)
import functools

import jax
import jax.numpy as jnp
from jax import lax
from jax.experimental import pallas as pl
from jax.experimental.pallas import tpu as pltpu
from jax.experimental.pallas import tpu_sc as plsc

N = 10000
E = 320000
D = 128
H = 64
R = 8
G = 64
C = 16
NB = 30
RN = R * N
HP = 128

BN = 1000
NBLK = N // BN
CH = 256
NCH = E // CH
CH2 = 160
NCH2 = E // CH2
NW = 32
NPAD = 10112
RPT = NPAD // 16
CPAD = 640 * 128
CPS = CPAD // 16
ZC = CPS // 4
ZR = 32
SROWS = RN // NW


def _k1_body(x_ref, wrel_ref, wroot_ref, y_ref, z_ref):
    xb = x_ref[...]
    y_ref[:, :H] = jnp.dot(xb, wrel_ref[...], preferred_element_type=jnp.float32)
    y_ref[:, H:] = jnp.zeros((BN, HP - H), jnp.float32)
    z_ref[...] = jnp.dot(xb, wroot_ref[...], preferred_element_type=jnp.float32)


def _k1(x, W_rel, W_root):
    return pl.pallas_call(
        _k1_body,
        grid=(NBLK,),
        in_specs=[
            pl.BlockSpec((BN, D), lambda i: (i, 0)),
            pl.BlockSpec((D, H), lambda i: (0, 0)),
            pl.BlockSpec((D, H), lambda i: (0, 0)),
        ],
        out_specs=[
            pl.BlockSpec((BN, HP), lambda i: (i, 0)),
            pl.BlockSpec((BN, H), lambda i: (i, 0)),
        ],
        out_shape=[
            jax.ShapeDtypeStruct((N, HP), jnp.float32),
            jax.ShapeDtypeStruct((N, H), jnp.float32),
        ],
    )(x, W_rel, W_root)


def _sc1_body(y_hbm, src_hbm, dst_hbm, w_hbm, t_hbm,
              agg_out, cnt_out,
              src_v, dst_v, w_v, t_v, seg_v, ones_v, rows_v, zc_v, zbuf_v,
              sem, sem2, sem3, sem4, agg_sh, cnt_sh):
    cid = lax.axis_index("c")
    sid = lax.axis_index("s")
    wid = sid * 2 + cid

    def _ones(i, _):
        ones_v[pl.ds(i * 16, 16)] = jnp.ones((16,), jnp.float32)
        return 0
    lax.fori_loop(0, CH // 16, _ones, 0)

    def _zrow(i, _):
        for c in range(HP // 16):
            zbuf_v[i, pl.ds(c * 16, 16)] = jnp.zeros((16,), jnp.float32)
        return 0
    lax.fori_loop(0, ZR, _zrow, 0)

    def _zc(i, _):
        zc_v[pl.ds(i * 16, 16)] = jnp.zeros((16,), jnp.float32)
        return 0
    lax.fori_loop(0, ZC // 16, _zc, 0)

    for k in range(RPT // ZR):
        pltpu.sync_copy(zbuf_v, agg_sh.at[pl.ds(sid * RPT + k * ZR, ZR)])
    if RPT % ZR:
        pltpu.sync_copy(zbuf_v.at[pl.ds(0, RPT % ZR)],
                        agg_sh.at[pl.ds(sid * RPT + (RPT // ZR) * ZR, RPT % ZR)])
    for k in range(4):
        pltpu.sync_copy(zc_v, cnt_sh.at[pl.ds(sid * CPS + k * ZC, ZC)])
    plsc.subcore_barrier()

    def _chunk(j, _):
        base = (wid + j * NW) * CH
        c0 = pltpu.async_copy(src_hbm.at[pl.ds(base, CH)], src_v, sem)
        c1 = pltpu.async_copy(dst_hbm.at[pl.ds(base, CH)], dst_v, sem2)
        c2 = pltpu.async_copy(w_hbm.at[pl.ds(base, CH)], w_v, sem3)
        c3 = pltpu.async_copy(t_hbm.at[pl.ds(base, CH)], t_v, sem4)
        c0.wait()
        c1.wait()
        c2.wait()
        c3.wait()
        pltpu.async_copy(y_hbm.at[src_v], rows_v, sem).wait()

        for eb in range(CH // 16):
            sl16 = pl.ds(eb * 16, 16)
            seg_v[sl16] = t_v[sl16] * N + dst_v[sl16]
            w16 = w_v[sl16]
            for l in range(16):
                e = eb * 16 + l
                wspl = w16.at[jnp.full((16,), l, jnp.int32)].get(
                    mode="promise_in_bounds")
                for c in range(H // 16):
                    sl = pl.ds(c * 16, 16)
                    rows_v[e, sl] = rows_v[e, sl] * wspl

        pltpu.sync_copy(rows_v, agg_sh.at[dst_v], add=True)
        pltpu.sync_copy(ones_v, cnt_sh.at[seg_v], add=True)
        return 0
    nch = (NCH // NW) + jnp.where(wid < NCH - (NCH // NW) * NW, 1, 0)
    lax.fori_loop(0, nch, _chunk, 0)
    plsc.subcore_barrier()

    pltpu.sync_copy(agg_sh.at[pl.ds(sid * RPT, RPT)],
                    agg_out.at[pl.ds(cid * NPAD + sid * RPT, RPT)])
    pltpu.sync_copy(cnt_sh.at[pl.ds(sid * CPS, CPS)],
                    cnt_out.at[pl.ds(cid * CPAD + sid * CPS, CPS)])


def _sc1(y, src, dst, w, t):
    mesh = plsc.VectorSubcoreMesh(core_axis_name="c", subcore_axis_name="s")
    f = functools.partial(
        pl.kernel,
        mesh=mesh,
        out_type=[
            jax.ShapeDtypeStruct((2 * NPAD, HP), jnp.float32),
            jax.ShapeDtypeStruct((2 * CPAD,), jnp.float32),
        ],
        scratch_types=[
            pltpu.VMEM((CH,), jnp.int32),
            pltpu.VMEM((CH,), jnp.int32),
            pltpu.VMEM((CH,), jnp.float32),
            pltpu.VMEM((CH,), jnp.int32),
            pltpu.VMEM((CH,), jnp.int32),
            pltpu.VMEM((CH,), jnp.float32),
            pltpu.VMEM((CH, HP), jnp.float32),
            pltpu.VMEM((ZC,), jnp.float32),
            pltpu.VMEM((ZR, HP), jnp.float32),
            pltpu.SemaphoreType.DMA,
            pltpu.SemaphoreType.DMA,
            pltpu.SemaphoreType.DMA,
            pltpu.SemaphoreType.DMA,
            pltpu.VMEM_SHARED((NPAD, HP), jnp.float32),
            pltpu.VMEM_SHARED((CPAD,), jnp.float32),
        ],
    )(_sc1_body)
    return f(y, src, dst, w, t)


BIC = CPAD // NBLK


def _kw_body(comp_ref, basisf_ref, wf_ref):
    wf_ref[...] = jnp.dot(comp_ref[...], basisf_ref[...],
                          preferred_element_type=jnp.float32)


def _kw(comp, basis_f):
    return pl.pallas_call(
        _kw_body,
        out_shape=jax.ShapeDtypeStruct((R, H * H), jnp.float32),
    )(comp, basis_f)


def _k3_body(cntc_ref, aggp_ref, z_ref, brel_ref, wstk_ref,
             root_ref, p_ref, q_ref, ic2_ref):
    ic_col = 1.0 / jnp.maximum(cntc_ref[0] + cntc_ref[1], 1.0)
    ic2_ref[...] = jnp.broadcast_to(ic_col, (BIC, HP))
    agg = aggp_ref[0, :, :H] + aggp_ref[1, :, :H]
    outb = jax.nn.relu(agg + brel_ref[...] + z_ref[...])
    q_ref[...] = jnp.dot(outb, root_ref[...], preferred_element_type=jnp.float32)
    for r in range(R):
        wr = wstk_ref[r * H:(r + 1) * H, :]
        p_ref[r, :, :H] = jnp.dot(outb, wr, preferred_element_type=jnp.float32)
        p_ref[r, :, H:] = jnp.zeros((BN, HP - H), jnp.float32)


def _k3(cntc, aggp, z, b_rel, Wstk, rgcn_root):
    return pl.pallas_call(
        _k3_body,
        grid=(NBLK,),
        in_specs=[
            pl.BlockSpec((2, BIC, 1), lambda i: (0, i, 0)),
            pl.BlockSpec((2, BN, HP), lambda i: (0, i, 0)),
            pl.BlockSpec((BN, H), lambda i: (i, 0)),
            pl.BlockSpec((1, H), lambda i: (0, 0)),
            pl.BlockSpec((R * H, H), lambda i: (0, 0)),
            pl.BlockSpec((H, H), lambda i: (0, 0)),
        ],
        out_specs=[
            pl.BlockSpec((R, BN, HP), lambda i: (0, i, 0)),
            pl.BlockSpec((BN, H), lambda i: (i, 0)),
            pl.BlockSpec((BIC, HP), lambda i: (i, 0)),
        ],
        out_shape=[
            jax.ShapeDtypeStruct((R, N, HP), jnp.float32),
            jax.ShapeDtypeStruct((N, H), jnp.float32),
            jax.ShapeDtypeStruct((CPAD, HP), jnp.float32),
        ],
    )(cntc, aggp, z, b_rel.reshape(1, H), Wstk, rgcn_root)


def _sc2_body(p_hbm, ic2_hbm, src_hbm, dst_hbm, t_hbm,
              acc_out,
              src_v, dst_v, t_v, gidx_v, seg_v, rows_v, icr_v, zbuf_v,
              sem, sem2, sem3, acc_sh):
    cid = lax.axis_index("c")
    sid = lax.axis_index("s")
    wid = sid * 2 + cid

    def _zrow(i, _):
        for c in range(HP // 16):
            zbuf_v[i, pl.ds(c * 16, 16)] = jnp.zeros((16,), jnp.float32)
        return 0
    lax.fori_loop(0, ZR, _zrow, 0)

    for k in range(RPT // ZR):
        pltpu.sync_copy(zbuf_v, acc_sh.at[pl.ds(sid * RPT + k * ZR, ZR)])
    if RPT % ZR:
        pltpu.sync_copy(zbuf_v.at[pl.ds(0, RPT % ZR)],
                        acc_sh.at[pl.ds(sid * RPT + (RPT // ZR) * ZR, RPT % ZR)])
    plsc.subcore_barrier()

    def _chunk(j, _):
        base = (wid + j * NW) * CH2
        c0 = pltpu.async_copy(src_hbm.at[pl.ds(base, CH2)], src_v, sem)
        c1 = pltpu.async_copy(dst_hbm.at[pl.ds(base, CH2)], dst_v, sem2)
        c2 = pltpu.async_copy(t_hbm.at[pl.ds(base, CH2)], t_v, sem3)
        c0.wait()
        c1.wait()
        c2.wait()

        def _idx16(eb, _):
            sl = pl.ds(eb * 16, 16)
            tn = t_v[sl] * N
            gidx_v[sl] = tn + src_v[sl]
            seg_v[sl] = tn + dst_v[sl]
            return 0
        lax.fori_loop(0, CH2 // 16, _idx16, 0)

        g0 = pltpu.async_copy(p_hbm.at[gidx_v], rows_v, sem)
        g1 = pltpu.async_copy(ic2_hbm.at[seg_v], icr_v, sem2)
        g0.wait()
        g1.wait()

        def _mul(e, _):
            for c in range(H // 16):
                sl = pl.ds(c * 16, 16)
                rows_v[e, sl] = rows_v[e, sl] * icr_v[e, sl]
            return 0
        lax.fori_loop(0, CH2, _mul, 0)

        pltpu.sync_copy(rows_v, acc_sh.at[dst_v], add=True)
        return 0
    nch = (NCH2 // NW) + jnp.where(wid < NCH2 - (NCH2 // NW) * NW, 1, 0)
    lax.fori_loop(0, nch, _chunk, 0)
    plsc.subcore_barrier()

    pltpu.sync_copy(acc_sh.at[pl.ds(sid * RPT, RPT)],
                    acc_out.at[pl.ds(cid * NPAD + sid * RPT, RPT)])


def _sc2(P, IC2, src, dst, t):
    mesh = plsc.VectorSubcoreMesh(core_axis_name="c", subcore_axis_name="s")
    f = functools.partial(
        pl.kernel,
        mesh=mesh,
        out_type=jax.ShapeDtypeStruct((2 * NPAD, HP), jnp.float32),
        scratch_types=[
            pltpu.VMEM((CH2,), jnp.int32),
            pltpu.VMEM((CH2,), jnp.int32),
            pltpu.VMEM((CH2,), jnp.int32),
            pltpu.VMEM((CH2,), jnp.int32),
            pltpu.VMEM((CH2,), jnp.int32),
            pltpu.VMEM((CH2, HP), jnp.float32),
            pltpu.VMEM((CH2, HP), jnp.float32),
            pltpu.VMEM((ZR, HP), jnp.float32),
            pltpu.SemaphoreType.DMA,
            pltpu.SemaphoreType.DMA,
            pltpu.SemaphoreType.DMA,
            pltpu.VMEM_SHARED((NPAD, HP), jnp.float32),
        ],
    )(_sc2_body)
    return f(P, IC2, src, dst, t)


def _k5_body(x_ref, accp_ref, q_ref, bias_ref, gb_ref,
             wl_ref, bl_ref, wf_ref, bf_ref, out_ref,
             sumx, maxx, sumo, maxo):
    i = pl.program_id(0)

    @pl.when(i == 0)
    def _init():
        sumx[...] = jnp.zeros_like(sumx)
        maxx[...] = jnp.full_like(maxx, -jnp.inf)
        sumo[...] = jnp.zeros_like(sumo)
        maxo[...] = jnp.full_like(maxo, -jnp.inf)

    acc = accp_ref[0, :, :H] + accp_ref[1, :, :H]
    out2 = jax.nn.relu(acc + q_ref[...] + bias_ref[...])

    xb = x_ref[...]
    gb = gb_ref[0]
    glo = jnp.min(gb)
    ghi = jnp.max(gb)

    def _seg(g, _):
        m = gb == g
        mx_x = jnp.max(jnp.where(m, xb, -jnp.inf), axis=0, keepdims=True)
        sm_x = jnp.sum(jnp.where(m, xb, 0.0), axis=0, keepdims=True)
        mx_o = jnp.max(jnp.where(m, out2, -jnp.inf), axis=0, keepdims=True)
        sm_o = jnp.sum(jnp.where(m, out2, 0.0), axis=0, keepdims=True)
        sl = pl.ds(g, 1)
        maxx[sl, :] = jnp.maximum(maxx[sl, :], mx_x)
        sumx[sl, :] = sumx[sl, :] + sm_x
        maxo[sl, :] = jnp.maximum(maxo[sl, :], mx_o)
        sumo[sl, :] = sumo[sl, :] + sm_o
        return 0
    lax.fori_loop(glo, ghi + 1, _seg, 0)

    @pl.when(i == NBLK - 1)
    def _final():
        hidden = jax.nn.relu(
            jnp.dot(sumx[...], wl_ref[0:D, :], preferred_element_type=jnp.float32)
            + jnp.dot(sumo[...], wl_ref[D:D + H, :], preferred_element_type=jnp.float32)
            + jnp.dot(maxx[...], wl_ref[D + H:2 * D + H, :], preferred_element_type=jnp.float32)
            + jnp.dot(maxo[...], wl_ref[2 * D + H:, :], preferred_element_type=jnp.float32)
            + bl_ref[...])
        logits = jnp.dot(hidden, wf_ref[...], preferred_element_type=jnp.float32) + bf_ref[...]
        mx = jnp.max(logits, axis=-1, keepdims=True)
        lse = jnp.log(jnp.sum(jnp.exp(logits - mx), axis=-1, keepdims=True))
        out_ref[...] = logits - mx - lse


def _k5(x, accp, q, rgcn_bias, graph_batch, W_lin, b_lin, W_fc, b_fc):
    return pl.pallas_call(
        _k5_body,
        grid=(NBLK,),
        in_specs=[
            pl.BlockSpec((BN, D), lambda i: (i, 0)),
            pl.BlockSpec((2, BN, HP), lambda i: (0, i, 0)),
            pl.BlockSpec((BN, H), lambda i: (i, 0)),
            pl.BlockSpec((1, H), lambda i: (0, 0)),
            pl.BlockSpec((1, BN, 1), lambda i: (i, 0, 0)),
            pl.BlockSpec((2 * (D + H), H), lambda i: (0, 0)),
            pl.BlockSpec((1, H), lambda i: (0, 0)),
            pl.BlockSpec((H, C), lambda i: (0, 0)),
            pl.BlockSpec((1, C), lambda i: (0, 0)),
        ],
        out_specs=pl.BlockSpec((G, C), lambda i: (0, 0)),
        out_shape=jax.ShapeDtypeStruct((G, C), jnp.float32),
        scratch_shapes=[
            pltpu.VMEM((G, D), jnp.float32),
            pltpu.VMEM((G, D), jnp.float32),
            pltpu.VMEM((G, H), jnp.float32),
            pltpu.VMEM((G, H), jnp.float32),
        ],
    )(x, accp, q, rgcn_bias.reshape(1, H),
      graph_batch.reshape(NBLK, BN, 1), W_lin, b_lin.reshape(1, H),
      W_fc, b_fc.reshape(1, C))


def kernel(x, edge_index, edge_weight, edge_type, graph_batch, W_rel, b_rel,
           W_root, basis, comp, rgcn_root, rgcn_bias, W_lin, b_lin, W_fc, b_fc):
    src = edge_index[0]
    dst = edge_index[1]

    y, z = _k1(x, W_rel, W_root)
    aggp, cntp = _sc1(y, src, dst, edge_weight, edge_type)
    Wf = _kw(comp, basis.reshape(NB, H * H))
    P, q, IC2 = _k3(cntp.reshape(2, CPAD, 1), aggp.reshape(2, NPAD, HP), z,
                    b_rel, Wf.reshape(R * H, H), rgcn_root)
    accp = _sc2(P.reshape(RN, HP), IC2, src, dst, edge_type)
    return _k5(x, accp.reshape(2, NPAD, HP), q, rgcn_bias, graph_batch,
               W_lin, b_lin, W_fc, b_fc)
